# Initial kernel scaffold; baseline (speedup 1.0000x reference)
import jax
import jax.numpy as jnp
from jax.experimental import pallas as pl


def kernel(hidden_tensor, router_w, w1_stack, w2_stack, shared_w1, shared_w2):
    raise NotImplementedError("write your pallas kernel here")


# speedup vs baseline: 1.8950x; 1.8950x over previous
import jax
import jax.numpy as jnp
from jax.experimental import pallas as pl
from jax.experimental.pallas import tpu as pltpu

N_EXPERTS = 8
TOP_K = 2
NEG_INF = -1e30


def _router_kernel(x_ref, rw_ref, wd_ref):
    x = x_ref[...]
    rw = rw_ref[...]
    logits = jax.lax.dot_general(
        x, rw, (((1,), (1,)), ((), ())),
        preferred_element_type=jnp.float32,
        precision=jax.lax.Precision.HIGHEST)
    T = logits.shape[0]
    lane8 = jax.lax.broadcasted_iota(jnp.int32, (T, N_EXPERTS), 1)
    m1 = jnp.max(logits, axis=1, keepdims=True)
    a1 = jnp.argmax(logits, axis=1)[:, None]
    masked = jnp.where(lane8 == a1, NEG_INF, logits)
    m2 = jnp.max(masked, axis=1, keepdims=True)
    a2 = jnp.argmax(masked, axis=1)[:, None]
    ex = jnp.exp(logits - m1)
    denom = jnp.sum(ex, axis=1, keepdims=True)
    p1 = 1.0 / denom
    p2 = jnp.exp(m2 - m1) / denom
    lane16 = jax.lax.broadcasted_iota(jnp.int32, (T, 16), 1)
    wd = jnp.where(lane16 == a1, p1, 0.0)
    wd = jnp.where(lane16 == a2, p2, wd)
    wd = jnp.where(lane16 == N_EXPERTS, 1.0, wd)
    wd_ref[...] = wd


def _moe_kernel(wd_ref, x_ref, w1_ref, w2_ref, o_ref):
    e = pl.program_id(1)
    x = x_ref[...]
    h = jax.lax.dot_general(
        x, w1_ref[...], (((1,), (1,)), ((), ())),
        preferred_element_type=jnp.float32)
    h = jnp.maximum(h, 0.0)
    h = h * h
    y = jax.lax.dot_general(
        h.astype(jnp.bfloat16), w2_ref[...], (((1,), (1,)), ((), ())),
        preferred_element_type=jnp.float32)
    wd = wd_ref[...]
    lane16 = jax.lax.broadcasted_iota(jnp.int32, wd.shape, 1)
    g = jnp.sum(jnp.where(lane16 == e, wd, 0.0), axis=1, keepdims=True)
    contrib = y * g

    @pl.when(e == 0)
    def _init():
        o_ref[...] = contrib

    @pl.when(e > 0)
    def _acc():
        o_ref[...] += contrib


def kernel(hidden_tensor, router_w, w1_stack, w2_stack, shared_w1, shared_w2):
    B, T, C = hidden_tensor.shape
    F = w1_stack.shape[1]
    x = hidden_tensor.reshape(T, C)

    wd = pl.pallas_call(
        _router_kernel,
        out_shape=jax.ShapeDtypeStruct((T, 16), jnp.float32),
    )(x, router_w)

    w1_all = jnp.concatenate([w1_stack, shared_w1[None]], axis=0)
    w2_all = jnp.concatenate([w2_stack, shared_w2[None]], axis=0)
    xb = x.astype(jnp.bfloat16)
    w1b = w1_all.astype(jnp.bfloat16)
    w2b = w2_all.astype(jnp.bfloat16)

    TB = 1024
    n_t = T // TB
    out = pl.pallas_call(
        _moe_kernel,
        grid=(n_t, N_EXPERTS + 1),
        in_specs=[
            pl.BlockSpec((TB, 16), lambda t, e: (t, 0)),
            pl.BlockSpec((TB, C), lambda t, e: (t, 0)),
            pl.BlockSpec((None, F, C), lambda t, e: (e, 0, 0)),
            pl.BlockSpec((None, C, F), lambda t, e: (e, 0, 0)),
        ],
        out_specs=pl.BlockSpec((TB, C), lambda t, e: (t, 0)),
        out_shape=jax.ShapeDtypeStruct((T, C), jnp.float32),
    )(wd, xb, w1b, w2b)

    return out.reshape(B, T, C)
